# baseline (speedup 1.0000x reference)
import functools

import jax
import jax.numpy as jnp
from jax import lax
from jax.experimental import pallas as pl
from jax.experimental.pallas import tpu as pltpu
from jax.experimental.pallas import tpu_sc as plsc

_C = 10
_IGN = _C - 1
_B, _H, _W = 16, 512, 512

_RSC = 160
_RTC = _H - _RSC
_BH = 88

_NW = 32
_L = 16

_ROWS_CW = _RSC // 2
_CCROWS = 8
_NCCHUNK = _ROWS_CW // _CCROWS

_PROWS_W = _RTC // 2
_P2ROWS = 16
_NP2 = _PROWS_W // _P2ROWS

_LN2 = 0.6931471805599453
_LOG2_POLY = (-0.07915036575313755, 0.6288157291847285, -2.081060203458998,
              4.028372766846473, -2.4967737679054225)


def _tc_body(pred_ref, tgt_ref, nll_ref, cnt_ref):
    b = pl.program_id(0)
    h = pl.program_id(1)

    @pl.when((b == 0) & (h == 0))
    def _():
        nll_ref[0, 0] = 0.0
        cnt_ref[0, 0] = 0.0

    t = tgt_ref[0]
    s = jnp.zeros(t.shape, jnp.float32)
    sel = jnp.zeros(t.shape, jnp.float32)
    for c in range(_C):
        x = pred_ref[0, c]
        s = s + jnp.exp(x)
        sel = jnp.where(t == c, x, sel)
    maskf = (t != _IGN).astype(jnp.float32)
    nll = (jnp.log(s) - sel) * maskf
    nll_ref[0, 0] += jnp.sum(nll)
    cnt_ref[0, 0] += jnp.sum(maskf)


def _tc_call(pred, target):
    grid = (_B, _RTC // _BH)
    return pl.pallas_call(
        _tc_body,
        grid=grid,
        in_specs=[
            pl.BlockSpec((1, _C, _BH, _W), lambda b, h: (b, 0, h, 0)),
            pl.BlockSpec((1, _BH, _W), lambda b, h: (b, h, 0)),
        ],
        out_specs=[
            pl.BlockSpec((1, 1), lambda b, h: (0, 0), memory_space=pltpu.SMEM),
            pl.BlockSpec((1, 1), lambda b, h: (0, 0), memory_space=pltpu.SMEM),
        ],
        out_shape=[
            jax.ShapeDtypeStruct((1, 1), jnp.float32),
            jax.ShapeDtypeStruct((1, 1), jnp.float32),
        ],
    )(pred, target)


@functools.partial(
    pl.kernel,
    out_type=[
        jax.ShapeDtypeStruct((_NW, _L), jnp.float32),
        jax.ShapeDtypeStruct((_NW, _L), jnp.float32),
        jax.ShapeDtypeStruct((_NW, _L), jnp.int32),
    ],
    mesh=plsc.VectorSubcoreMesh(core_axis_name="c", subcore_axis_name="s"),
    scratch_types=[
        pltpu.VMEM((_C, _CCROWS, _W), jnp.float32),
        pltpu.VMEM((_C, _CCROWS, _W), jnp.float32),
        pltpu.VMEM((_CCROWS, _W), jnp.int32),
        pltpu.VMEM((_CCROWS, _W), jnp.int32),
        pltpu.VMEM((_P2ROWS, _W), jnp.int32),
        pltpu.VMEM((_P2ROWS, _W), jnp.int32),
        pltpu.VMEM((_L,), jnp.float32),
        pltpu.VMEM((_L,), jnp.float32),
        pltpu.VMEM((_L,), jnp.int32),
        pltpu.SemaphoreType.DMA,
        pltpu.SemaphoreType.DMA,
    ],
)
def _sc_main(pred_hbm, tgt_hbm, nll_hbm, cnt_hbm, pres_hbm,
             bp0, bp1, bt0, bt1, pb0, pb1, nllv, cntv, presv, sem0, sem1):
    wid = lax.axis_index("s") * 2 + lax.axis_index("c")
    b = wid // 2
    r0 = _RTC + (wid % 2) * _ROWS_CW
    bps = (bp0, bp1)
    bts = (bt0, bt1)
    pbs = (pb0, pb1)
    sems = (sem0, sem1)

    def fire(k, which):
        cs = [pltpu.async_copy(
            pred_hbm.at[b, :, pl.ds(r0 + k * _CCROWS, _CCROWS), :],
            bps[which], sems[which])]
        cs.append(pltpu.async_copy(
            tgt_hbm.at[b, pl.ds(r0 + k * _CCROWS, _CCROWS), :],
            bts[which], sems[which]))
        return cs

    pr0 = (wid % 2) * _PROWS_W

    def pfire(k, which):
        return pltpu.async_copy(
            tgt_hbm.at[b, pl.ds(pr0 + k * _P2ROWS, _P2ROWS), :],
            pbs[which], sems[which])

    copies = [None, None]
    copies[0] = fire(0, 0)

    nll = jnp.zeros((_L,), jnp.float32)
    cnt = jnp.zeros((_L,), jnp.float32)
    pres = jnp.zeros((_L,), jnp.int32)
    one = jnp.ones((_L,), jnp.int32)
    zf = jnp.zeros((_L,), jnp.float32)
    onef = jnp.ones((_L,), jnp.float32)
    ign_v = jnp.full((_L,), _IGN, jnp.int32)
    cvecs = [jnp.full((_L,), c, jnp.int32) for c in range(_C)]
    c23 = jnp.full((_L,), 23, jnp.int32)
    c127 = jnp.full((_L,), 127, jnp.int32)
    cmant = jnp.full((_L,), 0x7FFFFF, jnp.int32)
    cone_f = jnp.full((_L,), 0x3F800000, jnp.int32)
    poly = [jnp.full((_L,), c, jnp.float32) for c in _LOG2_POLY]
    ln2_v = jnp.full((_L,), _LN2, jnp.float32)

    for i in range(_NCCHUNK):
        nxt = i + 1
        if nxt < _NCCHUNK:
            copies[nxt % 2] = fire(nxt, nxt % 2)
        else:
            copies[nxt % 2] = [pfire(0, nxt % 2)]
        for cp in copies[i % 2]:
            cp.wait()
        bp = bps[i % 2]
        bt = bts[i % 2]

        def row(r, carry, bp=bp, bt=bt):
            def col(cc, carry2):
                nll_a, cnt_a, pres_a = carry2
                t = bt[r, pl.ds(cc * _L, _L)]
                s = zf
                sel = zf
                for c in range(_C):
                    x = bp[c, r, pl.ds(cc * _L, _L)]
                    s = s + jnp.exp(x)
                    sel = jnp.where(t == cvecs[c], x, sel)
                bits = lax.bitcast_convert_type(s, jnp.int32)
                e = (bits >> c23) - c127
                mant = lax.bitcast_convert_type(
                    (bits & cmant) | cone_f, jnp.float32)
                p = poly[0]
                for coef in poly[1:]:
                    p = p * mant + coef
                ln_s = (p + e.astype(jnp.float32)) * ln2_v
                maskf = jnp.where(t != ign_v, onef, zf)
                nll_a = nll_a + (ln_s - sel) * maskf
                cnt_a = cnt_a + maskf
                pres_a = pres_a | (one << t)
                return (nll_a, cnt_a, pres_a)
            return lax.fori_loop(0, _W // _L, col, carry, unroll=2)

        nll, cnt, pres = lax.fori_loop(0, _CCROWS, row, (nll, cnt, pres))

    pcopies = [copies[_NCCHUNK % 2][0], None]
    for k in range(_NP2):
        nxt = k + 1
        if nxt < _NP2:
            pcopies[nxt % 2] = pfire(nxt, nxt % 2)
        pcopies[k % 2].wait()
        pb = pbs[k % 2]

        def prow(r, a, pb=pb):
            def pcol(cc, a2):
                v = pb[r, pl.ds(cc * _L, _L)]
                return a2 | (one << v)
            return lax.fori_loop(0, _W // _L, pcol, a, unroll=8)

        pres = lax.fori_loop(0, _P2ROWS, prow, pres)

    nllv[...] = nll
    cntv[...] = cnt
    presv[...] = pres
    pltpu.sync_copy(nllv, nll_hbm.at[wid])
    pltpu.sync_copy(cntv, cnt_hbm.at[wid])
    pltpu.sync_copy(presv, pres_hbm.at[wid])


def _combine_body(nll_tc_ref, cnt_tc_ref, nll_sc_ref, cnt_sc_ref, pres_ref,
                  out_ref):
    nll = nll_tc_ref[0, 0] + jnp.sum(nll_sc_ref[...])
    cnt = cnt_tc_ref[0, 0] + jnp.sum(cnt_sc_ref[...])
    w = pres_ref[...]
    nuniq = jnp.int32(0)
    for c in range(_C):
        nuniq = nuniq + jnp.max((w >> c) & 1)
    loss = nll / cnt
    out_ref[0, 0] = jnp.where(nuniq < 2, 0.0 * loss, loss)


def _combine_call(nll_tc, cnt_tc, nll_sc, cnt_sc, pres):
    return pl.pallas_call(
        _combine_body,
        in_specs=[
            pl.BlockSpec(memory_space=pltpu.SMEM),
            pl.BlockSpec(memory_space=pltpu.SMEM),
            pl.BlockSpec((_NW, _L), lambda: (0, 0)),
            pl.BlockSpec((_NW, _L), lambda: (0, 0)),
            pl.BlockSpec((_NW, _L), lambda: (0, 0)),
        ],
        out_specs=pl.BlockSpec(memory_space=pltpu.SMEM),
        out_shape=jax.ShapeDtypeStruct((1, 1), jnp.float32),
    )(nll_tc, cnt_tc, nll_sc, cnt_sc, pres)


def kernel(pred, target):
    nll_tc, cnt_tc = _tc_call(pred, target)
    nll_sc, cnt_sc, pres = _sc_main(pred, target)
    return _combine_call(nll_tc, cnt_tc, nll_sc, cnt_sc, pres)[0, 0]


# speedup vs baseline: 1.2032x; 1.2032x over previous
import functools

import jax
import jax.numpy as jnp
from jax import lax
from jax.experimental import pallas as pl
from jax.experimental.pallas import tpu as pltpu
from jax.experimental.pallas import tpu_sc as plsc

_C = 10
_IGN = _C - 1
_B, _H, _W = 16, 512, 512
_BH = 256

_KSC = 4
_BTC = _B - _KSC

_NW = 32
_L = 16

_WPB = _NW // _KSC
_ROWS_CW = _H // _WPB
_CCROWS = 8
_NCCHUNK = _ROWS_CW // _CCROWS

_PROWS_W = _BTC * _H // _NW
_P2ROWS = 32
_NP2 = _PROWS_W // _P2ROWS

_LN2 = 0.6931471805599453
_LOG2_POLY = (-0.07915036575313755, 0.6288157291847285, -2.081060203458998,
              4.028372766846473, -2.4967737679054225)


def _tc_body(pred_ref, tgt_ref, nll_ref, cnt_ref):
    b = pl.program_id(0)
    h = pl.program_id(1)

    @pl.when((b == 0) & (h == 0))
    def _():
        nll_ref[0, 0] = 0.0
        cnt_ref[0, 0] = 0.0

    t = tgt_ref[0]
    s = jnp.zeros(t.shape, jnp.float32)
    sel = jnp.zeros(t.shape, jnp.float32)
    for c in range(_C):
        x = pred_ref[0, c]
        s = s + jnp.exp(x)
        sel = jnp.where(t == c, x, sel)
    maskf = (t != _IGN).astype(jnp.float32)
    nll = (jnp.log(s) - sel) * maskf
    nll_ref[0, 0] += jnp.sum(nll)
    cnt_ref[0, 0] += jnp.sum(maskf)


def _tc_call(pred, target):
    grid = (_BTC, _H // _BH)
    return pl.pallas_call(
        _tc_body,
        grid=grid,
        in_specs=[
            pl.BlockSpec((1, _C, _BH, _W), lambda b, h: (b, 0, h, 0)),
            pl.BlockSpec((1, _BH, _W), lambda b, h: (b, h, 0)),
        ],
        out_specs=[
            pl.BlockSpec((1, 1), lambda b, h: (0, 0), memory_space=pltpu.SMEM),
            pl.BlockSpec((1, 1), lambda b, h: (0, 0), memory_space=pltpu.SMEM),
        ],
        out_shape=[
            jax.ShapeDtypeStruct((1, 1), jnp.float32),
            jax.ShapeDtypeStruct((1, 1), jnp.float32),
        ],
    )(pred, target)


@functools.partial(
    pl.kernel,
    out_type=[
        jax.ShapeDtypeStruct((_NW, _L), jnp.float32),
        jax.ShapeDtypeStruct((_NW, _L), jnp.float32),
        jax.ShapeDtypeStruct((_NW, _L), jnp.int32),
    ],
    mesh=plsc.VectorSubcoreMesh(core_axis_name="c", subcore_axis_name="s"),
    scratch_types=[
        pltpu.VMEM((_C, _CCROWS, _W), jnp.float32),
        pltpu.VMEM((_C, _CCROWS, _W), jnp.float32),
        pltpu.VMEM((_CCROWS, _W), jnp.int32),
        pltpu.VMEM((_CCROWS, _W), jnp.int32),
        pltpu.VMEM((_P2ROWS, _W), jnp.int32),
        pltpu.VMEM((_P2ROWS, _W), jnp.int32),
        pltpu.VMEM((_L,), jnp.float32),
        pltpu.VMEM((_L,), jnp.float32),
        pltpu.VMEM((_L,), jnp.int32),
        pltpu.SemaphoreType.DMA,
        pltpu.SemaphoreType.DMA,
    ],
)
def _sc_main(pred_hbm, tgt_hbm, nll_hbm, cnt_hbm, pres_hbm,
             bp0, bp1, bt0, bt1, pb0, pb1, nllv, cntv, presv, sem0, sem1):
    wid = lax.axis_index("s") * 2 + lax.axis_index("c")
    b = _BTC + wid // _WPB
    r0 = (wid % _WPB) * _ROWS_CW
    bps = (bp0, bp1)
    bts = (bt0, bt1)
    pbs = (pb0, pb1)
    sems = (sem0, sem1)

    def fire(k, which):
        cs = [pltpu.async_copy(
            pred_hbm.at[b, :, pl.ds(r0 + k * _CCROWS, _CCROWS), :],
            bps[which], sems[which])]
        cs.append(pltpu.async_copy(
            tgt_hbm.at[b, pl.ds(r0 + k * _CCROWS, _CCROWS), :],
            bts[which], sems[which]))
        return cs

    pg0 = wid * _PROWS_W

    def pfire(k, which):
        g = pg0 + k * _P2ROWS
        return pltpu.async_copy(
            tgt_hbm.at[g // _H, pl.ds(g % _H, _P2ROWS), :],
            pbs[which], sems[which])

    copies = [None, None]
    copies[0] = fire(0, 0)

    nll = jnp.zeros((_L,), jnp.float32)
    cnt = jnp.zeros((_L,), jnp.float32)
    pres = jnp.zeros((_L,), jnp.int32)
    one = jnp.ones((_L,), jnp.int32)
    zf = jnp.zeros((_L,), jnp.float32)
    onef = jnp.ones((_L,), jnp.float32)
    ign_v = jnp.full((_L,), _IGN, jnp.int32)
    cvecs = [jnp.full((_L,), c, jnp.int32) for c in range(_C)]
    c23 = jnp.full((_L,), 23, jnp.int32)
    c127 = jnp.full((_L,), 127, jnp.int32)
    cmant = jnp.full((_L,), 0x7FFFFF, jnp.int32)
    cone_f = jnp.full((_L,), 0x3F800000, jnp.int32)
    poly = [jnp.full((_L,), c, jnp.float32) for c in _LOG2_POLY]
    ln2_v = jnp.full((_L,), _LN2, jnp.float32)

    for i in range(_NCCHUNK):
        nxt = i + 1
        if nxt < _NCCHUNK:
            copies[nxt % 2] = fire(nxt, nxt % 2)
        else:
            copies[nxt % 2] = [pfire(0, nxt % 2)]
        for cp in copies[i % 2]:
            cp.wait()
        bp = bps[i % 2]
        bt = bts[i % 2]

        def row(r, carry, bp=bp, bt=bt):
            def col(cc, carry2):
                nll_a, cnt_a, pres_a = carry2
                t = bt[r, pl.ds(cc * _L, _L)]
                s = zf
                sel = zf
                for c in range(_C):
                    x = bp[c, r, pl.ds(cc * _L, _L)]
                    s = s + jnp.exp(x)
                    sel = jnp.where(t == cvecs[c], x, sel)
                bits = lax.bitcast_convert_type(s, jnp.int32)
                e = (bits >> c23) - c127
                mant = lax.bitcast_convert_type(
                    (bits & cmant) | cone_f, jnp.float32)
                p = poly[0]
                for coef in poly[1:]:
                    p = p * mant + coef
                ln_s = (p + e.astype(jnp.float32)) * ln2_v
                maskf = jnp.where(t != ign_v, onef, zf)
                nll_a = nll_a + (ln_s - sel) * maskf
                cnt_a = cnt_a + maskf
                pres_a = pres_a | (one << t)
                return (nll_a, cnt_a, pres_a)
            return lax.fori_loop(0, _W // _L, col, carry, unroll=2)

        nll, cnt, pres = lax.fori_loop(0, _CCROWS, row, (nll, cnt, pres))

    pcopies = [copies[_NCCHUNK % 2][0], None]
    for k in range(_NP2):
        nxt = k + 1
        if nxt < _NP2:
            pcopies[nxt % 2] = pfire(nxt, nxt % 2)
        pcopies[k % 2].wait()
        pb = pbs[k % 2]

        def prow(r, a, pb=pb):
            def pcol(cc, a2):
                v = pb[r, pl.ds(cc * _L, _L)]
                return a2 | (one << v)
            return lax.fori_loop(0, _W // _L, pcol, a, unroll=8)

        pres = lax.fori_loop(0, _P2ROWS, prow, pres)

    nllv[...] = nll
    cntv[...] = cnt
    presv[...] = pres
    pltpu.sync_copy(nllv, nll_hbm.at[wid])
    pltpu.sync_copy(cntv, cnt_hbm.at[wid])
    pltpu.sync_copy(presv, pres_hbm.at[wid])


def _combine_body(nll_tc_ref, cnt_tc_ref, nll_sc_ref, cnt_sc_ref, pres_ref,
                  out_ref):
    nll = nll_tc_ref[0, 0] + jnp.sum(nll_sc_ref[...])
    cnt = cnt_tc_ref[0, 0] + jnp.sum(cnt_sc_ref[...])
    w = pres_ref[...]
    nuniq = jnp.int32(0)
    for c in range(_C):
        nuniq = nuniq + jnp.max((w >> c) & 1)
    loss = nll / cnt
    out_ref[0, 0] = jnp.where(nuniq < 2, 0.0 * loss, loss)


def _combine_call(nll_tc, cnt_tc, nll_sc, cnt_sc, pres):
    return pl.pallas_call(
        _combine_body,
        in_specs=[
            pl.BlockSpec(memory_space=pltpu.SMEM),
            pl.BlockSpec(memory_space=pltpu.SMEM),
            pl.BlockSpec((_NW, _L), lambda: (0, 0)),
            pl.BlockSpec((_NW, _L), lambda: (0, 0)),
            pl.BlockSpec((_NW, _L), lambda: (0, 0)),
        ],
        out_specs=pl.BlockSpec(memory_space=pltpu.SMEM),
        out_shape=jax.ShapeDtypeStruct((1, 1), jnp.float32),
    )(nll_tc, cnt_tc, nll_sc, cnt_sc, pres)


def kernel(pred, target):
    nll_tc, cnt_tc = _tc_call(pred, target)
    nll_sc, cnt_sc, pres = _sc_main(pred, target)
    return _combine_call(nll_tc, cnt_tc, nll_sc, cnt_sc, pres)[0, 0]


# speedup vs baseline: 1.2166x; 1.0111x over previous
import functools

import jax
import jax.numpy as jnp
from jax import lax
from jax.experimental import pallas as pl
from jax.experimental.pallas import tpu as pltpu
from jax.experimental.pallas import tpu_sc as plsc

_C = 10
_IGN = _C - 1
_B, _H, _W = 16, 512, 512
_BH = 512

_KSC = 4
_BTC = _B - _KSC

_NW = 32
_L = 16

_WPB = _NW // _KSC
_ROWS_CW = _H // _WPB
_CCROWS = 8
_NCCHUNK = _ROWS_CW // _CCROWS

_PROWS_W = _BTC * _H // _NW
_P2ROWS = 32
_NP2 = _PROWS_W // _P2ROWS

_LN2 = 0.6931471805599453
_LOG2_POLY = (-0.07915036575313755, 0.6288157291847285, -2.081060203458998,
              4.028372766846473, -2.4967737679054225)


def _tc_body(pred_ref, tgt_ref, nll_ref, cnt_ref):
    b = pl.program_id(0)
    h = pl.program_id(1)

    @pl.when((b == 0) & (h == 0))
    def _():
        nll_ref[0, 0] = 0.0
        cnt_ref[0, 0] = 0.0

    t = tgt_ref[0]
    s = jnp.zeros(t.shape, jnp.float32)
    sel = jnp.zeros(t.shape, jnp.float32)
    for c in range(_C):
        x = pred_ref[0, c]
        s = s + jnp.exp(x)
        sel = jnp.where(t == c, x, sel)
    maskf = (t != _IGN).astype(jnp.float32)
    nll = (jnp.log(s) - sel) * maskf
    nll_ref[0, 0] += jnp.sum(nll)
    cnt_ref[0, 0] += jnp.sum(maskf)


def _tc_call(pred, target):
    grid = (_BTC, _H // _BH)
    return pl.pallas_call(
        _tc_body,
        grid=grid,
        in_specs=[
            pl.BlockSpec((1, _C, _BH, _W), lambda b, h: (b, 0, h, 0)),
            pl.BlockSpec((1, _BH, _W), lambda b, h: (b, h, 0)),
        ],
        out_specs=[
            pl.BlockSpec((1, 1), lambda b, h: (0, 0), memory_space=pltpu.SMEM),
            pl.BlockSpec((1, 1), lambda b, h: (0, 0), memory_space=pltpu.SMEM),
        ],
        out_shape=[
            jax.ShapeDtypeStruct((1, 1), jnp.float32),
            jax.ShapeDtypeStruct((1, 1), jnp.float32),
        ],
    )(pred, target)


@functools.partial(
    pl.kernel,
    out_type=[
        jax.ShapeDtypeStruct((_NW, _L), jnp.float32),
        jax.ShapeDtypeStruct((_NW, _L), jnp.float32),
        jax.ShapeDtypeStruct((_NW, _L), jnp.int32),
    ],
    mesh=plsc.VectorSubcoreMesh(core_axis_name="c", subcore_axis_name="s"),
    scratch_types=[
        pltpu.VMEM((_C, _CCROWS, _W), jnp.float32),
        pltpu.VMEM((_C, _CCROWS, _W), jnp.float32),
        pltpu.VMEM((_CCROWS, _W), jnp.int32),
        pltpu.VMEM((_CCROWS, _W), jnp.int32),
        pltpu.VMEM((_P2ROWS, _W), jnp.int32),
        pltpu.VMEM((_P2ROWS, _W), jnp.int32),
        pltpu.VMEM((_L,), jnp.float32),
        pltpu.VMEM((_L,), jnp.float32),
        pltpu.VMEM((_L,), jnp.int32),
        pltpu.SemaphoreType.DMA,
        pltpu.SemaphoreType.DMA,
    ],
)
def _sc_main(pred_hbm, tgt_hbm, nll_hbm, cnt_hbm, pres_hbm,
             bp0, bp1, bt0, bt1, pb0, pb1, nllv, cntv, presv, sem0, sem1):
    wid = lax.axis_index("s") * 2 + lax.axis_index("c")
    b = _BTC + wid // _WPB
    r0 = (wid % _WPB) * _ROWS_CW
    bps = (bp0, bp1)
    bts = (bt0, bt1)
    pbs = (pb0, pb1)
    sems = (sem0, sem1)

    def fire(k, which):
        cs = [pltpu.async_copy(
            pred_hbm.at[b, :, pl.ds(r0 + k * _CCROWS, _CCROWS), :],
            bps[which], sems[which])]
        cs.append(pltpu.async_copy(
            tgt_hbm.at[b, pl.ds(r0 + k * _CCROWS, _CCROWS), :],
            bts[which], sems[which]))
        return cs

    pg0 = wid * _PROWS_W

    def pfire(k, which):
        g = pg0 + k * _P2ROWS
        return pltpu.async_copy(
            tgt_hbm.at[g // _H, pl.ds(g % _H, _P2ROWS), :],
            pbs[which], sems[which])

    copies = [None, None]
    copies[0] = fire(0, 0)

    nll = jnp.zeros((_L,), jnp.float32)
    cnt = jnp.zeros((_L,), jnp.float32)
    pres = jnp.zeros((_L,), jnp.int32)
    one = jnp.ones((_L,), jnp.int32)
    zf = jnp.zeros((_L,), jnp.float32)
    onef = jnp.ones((_L,), jnp.float32)
    ign_v = jnp.full((_L,), _IGN, jnp.int32)
    cvecs = [jnp.full((_L,), c, jnp.int32) for c in range(_C)]
    c23 = jnp.full((_L,), 23, jnp.int32)
    c127 = jnp.full((_L,), 127, jnp.int32)
    cmant = jnp.full((_L,), 0x7FFFFF, jnp.int32)
    cone_f = jnp.full((_L,), 0x3F800000, jnp.int32)
    poly = [jnp.full((_L,), c, jnp.float32) for c in _LOG2_POLY]
    ln2_v = jnp.full((_L,), _LN2, jnp.float32)

    for i in range(_NCCHUNK):
        nxt = i + 1
        if nxt < _NCCHUNK:
            copies[nxt % 2] = fire(nxt, nxt % 2)
        else:
            copies[nxt % 2] = [pfire(0, nxt % 2)]
        for cp in copies[i % 2]:
            cp.wait()
        bp = bps[i % 2]
        bt = bts[i % 2]

        def row(r, carry, bp=bp, bt=bt):
            def col(cc, carry2):
                nll_a, cnt_a, pres_a = carry2
                t = bt[r, pl.ds(cc * _L, _L)]
                s = zf
                sel = zf
                for c in range(_C):
                    x = bp[c, r, pl.ds(cc * _L, _L)]
                    s = s + jnp.exp(x)
                    sel = jnp.where(t == cvecs[c], x, sel)
                bits = lax.bitcast_convert_type(s, jnp.int32)
                e = (bits >> c23) - c127
                mant = lax.bitcast_convert_type(
                    (bits & cmant) | cone_f, jnp.float32)
                p = poly[0]
                for coef in poly[1:]:
                    p = p * mant + coef
                ln_s = (p + e.astype(jnp.float32)) * ln2_v
                maskf = jnp.where(t != ign_v, onef, zf)
                nll_a = nll_a + (ln_s - sel) * maskf
                cnt_a = cnt_a + maskf
                pres_a = pres_a | (one << t)
                return (nll_a, cnt_a, pres_a)
            return lax.fori_loop(0, _W // _L, col, carry, unroll=2)

        nll, cnt, pres = lax.fori_loop(0, _CCROWS, row, (nll, cnt, pres))

    pcopies = [copies[_NCCHUNK % 2][0], None]
    for k in range(_NP2):
        nxt = k + 1
        if nxt < _NP2:
            pcopies[nxt % 2] = pfire(nxt, nxt % 2)
        pcopies[k % 2].wait()
        pb = pbs[k % 2]

        def prow(r, a, pb=pb):
            def pcol(cc, a2):
                v = pb[r, pl.ds(cc * _L, _L)]
                return a2 | (one << v)
            return lax.fori_loop(0, _W // _L, pcol, a, unroll=8)

        pres = lax.fori_loop(0, _P2ROWS, prow, pres)

    nllv[...] = nll
    cntv[...] = cnt
    presv[...] = pres
    pltpu.sync_copy(nllv, nll_hbm.at[wid])
    pltpu.sync_copy(cntv, cnt_hbm.at[wid])
    pltpu.sync_copy(presv, pres_hbm.at[wid])


def _combine_body(nll_tc_ref, cnt_tc_ref, nll_sc_ref, cnt_sc_ref, pres_ref,
                  out_ref):
    nll = nll_tc_ref[0, 0] + jnp.sum(nll_sc_ref[...])
    cnt = cnt_tc_ref[0, 0] + jnp.sum(cnt_sc_ref[...])
    w = pres_ref[...]
    nuniq = jnp.int32(0)
    for c in range(_C):
        nuniq = nuniq + jnp.max((w >> c) & 1)
    loss = nll / cnt
    out_ref[0, 0] = jnp.where(nuniq < 2, 0.0 * loss, loss)


def _combine_call(nll_tc, cnt_tc, nll_sc, cnt_sc, pres):
    return pl.pallas_call(
        _combine_body,
        in_specs=[
            pl.BlockSpec(memory_space=pltpu.SMEM),
            pl.BlockSpec(memory_space=pltpu.SMEM),
            pl.BlockSpec((_NW, _L), lambda: (0, 0)),
            pl.BlockSpec((_NW, _L), lambda: (0, 0)),
            pl.BlockSpec((_NW, _L), lambda: (0, 0)),
        ],
        out_specs=pl.BlockSpec(memory_space=pltpu.SMEM),
        out_shape=jax.ShapeDtypeStruct((1, 1), jnp.float32),
    )(nll_tc, cnt_tc, nll_sc, cnt_sc, pres)


def kernel(pred, target):
    nll_tc, cnt_tc = _tc_call(pred, target)
    nll_sc, cnt_sc, pres = _sc_main(pred, target)
    return _combine_call(nll_tc, cnt_tc, nll_sc, cnt_sc, pres)[0, 0]


# speedup vs baseline: 1.2309x; 1.0118x over previous
import functools

import jax
import jax.numpy as jnp
from jax import lax
from jax.experimental import pallas as pl
from jax.experimental.pallas import tpu as pltpu
from jax.experimental.pallas import tpu_sc as plsc

_C = 10
_IGN = _C - 1
_B, _H, _W = 16, 512, 512
_BH = 512

_KSC = 4
_BTC = _B - _KSC

_NW = 32
_L = 16

_WPB = _NW // _KSC
_ROWS_CW = _H // _WPB
_CCROWS = 8
_NCCHUNK = _ROWS_CW // _CCROWS

_PROWS_W = _BTC * _H // _NW
_P2ROWS = 32
_NP2 = _PROWS_W // _P2ROWS

_LN2 = 0.6931471805599453
_LOG2_POLY = (-0.07915036575313755, 0.6288157291847285, -2.081060203458998,
              4.028372766846473, -2.4967737679054225)


def _tc_body(pred_ref, tgt_ref, nll_ref, cnt_ref):
    b = pl.program_id(0)
    h = pl.program_id(1)

    @pl.when((b == 0) & (h == 0))
    def _():
        nll_ref[0, 0] = 0.0
        cnt_ref[0, 0] = 0.0

    for bb in range(2):
        t = tgt_ref[bb]
        s = jnp.zeros(t.shape, jnp.float32)
        sel = jnp.zeros(t.shape, jnp.float32)
        for c in range(_C):
            x = pred_ref[bb, c]
            s = s + jnp.exp(x)
            sel = jnp.where(t == c, x, sel)
        maskf = (t != _IGN).astype(jnp.float32)
        nll = (jnp.log(s) - sel) * maskf
        nll_ref[0, 0] += jnp.sum(nll)
        cnt_ref[0, 0] += jnp.sum(maskf)


def _tc_call(pred, target):
    grid = (_BTC // 2, _H // _BH)
    return pl.pallas_call(
        _tc_body,
        grid=grid,
        in_specs=[
            pl.BlockSpec((2, _C, _BH, _W), lambda b, h: (b, 0, h, 0)),
            pl.BlockSpec((2, _BH, _W), lambda b, h: (b, h, 0)),
        ],
        out_specs=[
            pl.BlockSpec((1, 1), lambda b, h: (0, 0), memory_space=pltpu.SMEM),
            pl.BlockSpec((1, 1), lambda b, h: (0, 0), memory_space=pltpu.SMEM),
        ],
        out_shape=[
            jax.ShapeDtypeStruct((1, 1), jnp.float32),
            jax.ShapeDtypeStruct((1, 1), jnp.float32),
        ],
    )(pred, target)


@functools.partial(
    pl.kernel,
    out_type=[
        jax.ShapeDtypeStruct((_NW, _L), jnp.float32),
        jax.ShapeDtypeStruct((_NW, _L), jnp.float32),
        jax.ShapeDtypeStruct((_NW, _L), jnp.int32),
    ],
    mesh=plsc.VectorSubcoreMesh(core_axis_name="c", subcore_axis_name="s"),
    scratch_types=[
        pltpu.VMEM((_C, _CCROWS, _W), jnp.float32),
        pltpu.VMEM((_C, _CCROWS, _W), jnp.float32),
        pltpu.VMEM((_CCROWS, _W), jnp.int32),
        pltpu.VMEM((_CCROWS, _W), jnp.int32),
        pltpu.VMEM((_P2ROWS, _W), jnp.int32),
        pltpu.VMEM((_P2ROWS, _W), jnp.int32),
        pltpu.VMEM((_L,), jnp.float32),
        pltpu.VMEM((_L,), jnp.float32),
        pltpu.VMEM((_L,), jnp.int32),
        pltpu.SemaphoreType.DMA,
        pltpu.SemaphoreType.DMA,
    ],
)
def _sc_main(pred_hbm, tgt_hbm, nll_hbm, cnt_hbm, pres_hbm,
             bp0, bp1, bt0, bt1, pb0, pb1, nllv, cntv, presv, sem0, sem1):
    wid = lax.axis_index("s") * 2 + lax.axis_index("c")
    b = _BTC + wid // _WPB
    r0 = (wid % _WPB) * _ROWS_CW
    bps = (bp0, bp1)
    bts = (bt0, bt1)
    pbs = (pb0, pb1)
    sems = (sem0, sem1)

    def fire(k, which):
        cs = [pltpu.async_copy(
            pred_hbm.at[b, :, pl.ds(r0 + k * _CCROWS, _CCROWS), :],
            bps[which], sems[which])]
        cs.append(pltpu.async_copy(
            tgt_hbm.at[b, pl.ds(r0 + k * _CCROWS, _CCROWS), :],
            bts[which], sems[which]))
        return cs

    pg0 = wid * _PROWS_W

    def pfire(k, which):
        g = pg0 + k * _P2ROWS
        return pltpu.async_copy(
            tgt_hbm.at[g // _H, pl.ds(g % _H, _P2ROWS), :],
            pbs[which], sems[which])

    copies = [None, None]
    copies[0] = fire(0, 0)

    nll = jnp.zeros((_L,), jnp.float32)
    cnt = jnp.zeros((_L,), jnp.float32)
    pres = jnp.zeros((_L,), jnp.int32)
    one = jnp.ones((_L,), jnp.int32)
    zf = jnp.zeros((_L,), jnp.float32)
    onef = jnp.ones((_L,), jnp.float32)
    ign_v = jnp.full((_L,), _IGN, jnp.int32)
    cvecs = [jnp.full((_L,), c, jnp.int32) for c in range(_C)]
    c23 = jnp.full((_L,), 23, jnp.int32)
    c127 = jnp.full((_L,), 127, jnp.int32)
    cmant = jnp.full((_L,), 0x7FFFFF, jnp.int32)
    cone_f = jnp.full((_L,), 0x3F800000, jnp.int32)
    poly = [jnp.full((_L,), c, jnp.float32) for c in _LOG2_POLY]
    ln2_v = jnp.full((_L,), _LN2, jnp.float32)

    for i in range(_NCCHUNK):
        nxt = i + 1
        if nxt < _NCCHUNK:
            copies[nxt % 2] = fire(nxt, nxt % 2)
        else:
            copies[nxt % 2] = [pfire(0, nxt % 2)]
        for cp in copies[i % 2]:
            cp.wait()
        bp = bps[i % 2]
        bt = bts[i % 2]

        def row(r, carry, bp=bp, bt=bt):
            def col(cc, carry2):
                nll_a, cnt_a, pres_a = carry2
                t = bt[r, pl.ds(cc * _L, _L)]
                s = zf
                sel = zf
                for c in range(_C):
                    x = bp[c, r, pl.ds(cc * _L, _L)]
                    s = s + jnp.exp(x)
                    sel = jnp.where(t == cvecs[c], x, sel)
                bits = lax.bitcast_convert_type(s, jnp.int32)
                e = (bits >> c23) - c127
                mant = lax.bitcast_convert_type(
                    (bits & cmant) | cone_f, jnp.float32)
                p = poly[0]
                for coef in poly[1:]:
                    p = p * mant + coef
                ln_s = (p + e.astype(jnp.float32)) * ln2_v
                maskf = jnp.where(t != ign_v, onef, zf)
                nll_a = nll_a + (ln_s - sel) * maskf
                cnt_a = cnt_a + maskf
                pres_a = pres_a | (one << t)
                return (nll_a, cnt_a, pres_a)
            return lax.fori_loop(0, _W // _L, col, carry, unroll=2)

        nll, cnt, pres = lax.fori_loop(0, _CCROWS, row, (nll, cnt, pres))

    pcopies = [copies[_NCCHUNK % 2][0], None]
    for k in range(_NP2):
        nxt = k + 1
        if nxt < _NP2:
            pcopies[nxt % 2] = pfire(nxt, nxt % 2)
        pcopies[k % 2].wait()
        pb = pbs[k % 2]

        def prow(r, a, pb=pb):
            def pcol(cc, a2):
                v = pb[r, pl.ds(cc * _L, _L)]
                return a2 | (one << v)
            return lax.fori_loop(0, _W // _L, pcol, a, unroll=8)

        pres = lax.fori_loop(0, _P2ROWS, prow, pres)

    nllv[...] = nll
    cntv[...] = cnt
    presv[...] = pres
    pltpu.sync_copy(nllv, nll_hbm.at[wid])
    pltpu.sync_copy(cntv, cnt_hbm.at[wid])
    pltpu.sync_copy(presv, pres_hbm.at[wid])


def _combine_body(nll_tc_ref, cnt_tc_ref, nll_sc_ref, cnt_sc_ref, pres_ref,
                  out_ref):
    nll = nll_tc_ref[0, 0] + jnp.sum(nll_sc_ref[...])
    cnt = cnt_tc_ref[0, 0] + jnp.sum(cnt_sc_ref[...])
    w = pres_ref[...]
    nuniq = jnp.int32(0)
    for c in range(_C):
        nuniq = nuniq + jnp.max((w >> c) & 1)
    loss = nll / cnt
    out_ref[0, 0] = jnp.where(nuniq < 2, 0.0 * loss, loss)


def _combine_call(nll_tc, cnt_tc, nll_sc, cnt_sc, pres):
    return pl.pallas_call(
        _combine_body,
        in_specs=[
            pl.BlockSpec(memory_space=pltpu.SMEM),
            pl.BlockSpec(memory_space=pltpu.SMEM),
            pl.BlockSpec((_NW, _L), lambda: (0, 0)),
            pl.BlockSpec((_NW, _L), lambda: (0, 0)),
            pl.BlockSpec((_NW, _L), lambda: (0, 0)),
        ],
        out_specs=pl.BlockSpec(memory_space=pltpu.SMEM),
        out_shape=jax.ShapeDtypeStruct((1, 1), jnp.float32),
    )(nll_tc, cnt_tc, nll_sc, cnt_sc, pres)


def kernel(pred, target):
    nll_tc, cnt_tc = _tc_call(pred, target)
    nll_sc, cnt_sc, pres = _sc_main(pred, target)
    return _combine_call(nll_tc, cnt_tc, nll_sc, cnt_sc, pres)[0, 0]
